# baseline (speedup 1.0000x reference)
import jax
import jax.numpy as jnp
from jax.experimental import pallas as pl
from jax.experimental.pallas import tpu as pltpu

_BM1 = 400
_BM = 400
_NK = 1
_BKP = 10240
_S = 4
_BW = _BKP // _S


def _p1_kernel(adj_ref, xb_ref, abf_ref, t1_ref):
    n = adj_ref.shape[1]
    ab = adj_ref[...].astype(jnp.bfloat16)
    abf_ref[:, :n] = ab
    abf_ref[:, n:] = jnp.zeros((abf_ref.shape[0], abf_ref.shape[1] - n),
                               jnp.bfloat16)
    t1_ref[...] = jnp.dot(
        ab, xb_ref[...], preferred_element_type=jnp.float32
    ).astype(jnp.bfloat16)


def _fetch_once(hbm_ref, vmem_ref, sem):
    @pl.when((pl.program_id(0) == 0) & (pl.program_id(1) == 0))
    def _():
        n = hbm_ref.shape[0]
        np_ = vmem_ref.shape[0]
        vmem_ref[pl.ds(n, np_ - n), :] = jnp.zeros(
            (np_ - n, vmem_ref.shape[1]), vmem_ref.dtype)
        cp = pltpu.make_async_copy(hbm_ref, vmem_ref.at[pl.ds(0, n), :], sem)
        cp.start()
        cp.wait()


def _partial(a_refs, vf, k, acc):
    part = jnp.dot(
        a_refs[0][...], vf[pl.ds(k * _BKP, _BW), :],
        preferred_element_type=jnp.float32
    )
    for j in range(1, _S):
        part = part + jnp.dot(
            a_refs[j][...], vf[pl.ds(k * _BKP + j * _BW, _BW), :],
            preferred_element_type=jnp.float32
        )

    @pl.when(k == 0)
    def _():
        acc[...] = part

    @pl.when(k > 0)
    def _():
        acc[...] = acc[...] + part


def _ax_kernel(vf_hbm, *rest):
    a_refs, (o_ref, vf, acc, sem) = rest[:_S], rest[_S:]
    _fetch_once(vf_hbm, vf, sem)
    k = pl.program_id(1)
    _partial(a_refs, vf, k, acc)

    @pl.when(k == _NK - 1)
    def _():
        o_ref[...] = acc[...].astype(jnp.bfloat16)


def _p2_kernel(t1_hbm, *rest):
    a_refs = rest[:_S]
    x_ref, xb_ref, w_ref, b_ref, hf_ref, hb_ref, t1f, acc, sem = rest[_S:]
    _fetch_once(t1_hbm, t1f, sem)
    i, k = pl.program_id(0), pl.program_id(1)
    _partial(a_refs, t1f, k, acc)

    @pl.when(k == _NK - 1)
    def _():
        t1_blk = t1f[pl.ds(i * _BM, _BM), :]
        tx2 = 2.0 * acc[...] - x_ref[...]
        h = (
            jnp.dot(xb_ref[...], w_ref[0, :, :], preferred_element_type=jnp.float32)
            + jnp.dot(t1_blk, w_ref[1, :, :], preferred_element_type=jnp.float32)
            + jnp.dot(tx2.astype(jnp.bfloat16), w_ref[2, :, :],
                      preferred_element_type=jnp.float32)
            + b_ref[...]
        )
        h = jnp.maximum(h, 0.0)
        hf_ref[...] = h
        hb_ref[...] = h.astype(jnp.bfloat16)


def _p4_kernel(t1_hbm, *rest):
    a_refs = rest[:_S]
    hf_ref, hb_ref, w_ref, b_ref, o_ref, t1f, acc, sem = rest[_S:]
    _fetch_once(t1_hbm, t1f, sem)
    i, k = pl.program_id(0), pl.program_id(1)
    _partial(a_refs, t1f, k, acc)

    @pl.when(k == _NK - 1)
    def _():
        t1_blk = t1f[pl.ds(i * _BM, _BM), :]
        th2 = 2.0 * acc[...] - hf_ref[...]
        logits = (
            jnp.dot(hb_ref[...], w_ref[0, :, :], preferred_element_type=jnp.float32)
            + jnp.dot(t1_blk, w_ref[1, :, :], preferred_element_type=jnp.float32)
            + jnp.dot(th2.astype(jnp.bfloat16), w_ref[2, :, :],
                      preferred_element_type=jnp.float32)
            + b_ref[...]
        )
        m = jnp.max(logits, axis=1, keepdims=True)
        e = logits - m
        o_ref[...] = e - jnp.log(jnp.sum(jnp.exp(e), axis=1, keepdims=True))


def _params(n_dims):
    return pltpu.CompilerParams(dimension_semantics=("arbitrary",) * n_dims)


def kernel(x, adj, W1, b1, W2, b2):
    N, F = x.shape
    H = W1.shape[2]
    C = W2.shape[2]
    NP = _NK * _BKP
    xb = x.astype(jnp.bfloat16)
    W1b = W1.astype(jnp.bfloat16)
    W2b = W2.astype(jnp.bfloat16)
    b1r = b1.reshape(1, H)
    b2r = b2.reshape(1, C)
    grid2 = (N // _BM, _NK)

    astreams = [
        pl.BlockSpec((_BM, _BW), (lambda j: (lambda i, k: (i, _S * k + j)))(j))
        for j in range(_S)
    ]
    mrow = lambda i, k: (i, 0)
    const2 = lambda i, k: (0, 0)
    hbm = pl.BlockSpec(memory_space=pl.ANY)

    abf, t1 = pl.pallas_call(
        _p1_kernel,
        grid=(N // _BM1,),
        in_specs=[
            pl.BlockSpec((_BM1, N), lambda i: (i, 0)),
            pl.BlockSpec((N, F), lambda i: (0, 0)),
        ],
        out_specs=[
            pl.BlockSpec((_BM1, NP), lambda i: (i, 0)),
            pl.BlockSpec((_BM1, F), lambda i: (i, 0)),
        ],
        out_shape=[
            jax.ShapeDtypeStruct((N, NP), jnp.bfloat16),
            jax.ShapeDtypeStruct((N, F), jnp.bfloat16),
        ],
        compiler_params=_params(1),
    )(adj, xb)

    hf, hb = pl.pallas_call(
        _p2_kernel,
        grid=grid2,
        in_specs=[
            hbm,
            *astreams,
            pl.BlockSpec((_BM, F), mrow),
            pl.BlockSpec((_BM, F), mrow),
            pl.BlockSpec((3, F, H), lambda i, k: (0, 0, 0)),
            pl.BlockSpec((1, H), const2),
        ],
        out_specs=[
            pl.BlockSpec((_BM, H), mrow),
            pl.BlockSpec((_BM, H), mrow),
        ],
        out_shape=[
            jax.ShapeDtypeStruct((N, H), jnp.float32),
            jax.ShapeDtypeStruct((N, H), jnp.bfloat16),
        ],
        scratch_shapes=[
            pltpu.VMEM((NP, F), jnp.bfloat16),
            pltpu.VMEM((_BM, H), jnp.float32),
            pltpu.SemaphoreType.DMA,
        ],
        compiler_params=_params(2),
    )(t1, *([abf] * _S), x, xb, W1b, b1r)

    th1 = pl.pallas_call(
        _ax_kernel,
        grid=grid2,
        in_specs=[
            hbm,
            *astreams,
        ],
        out_specs=pl.BlockSpec((_BM, H), mrow),
        out_shape=jax.ShapeDtypeStruct((N, H), jnp.bfloat16),
        scratch_shapes=[
            pltpu.VMEM((NP, H), jnp.bfloat16),
            pltpu.VMEM((_BM, H), jnp.float32),
            pltpu.SemaphoreType.DMA,
        ],
        compiler_params=_params(2),
    )(hb, *([abf] * _S))

    out = pl.pallas_call(
        _p4_kernel,
        grid=grid2,
        in_specs=[
            hbm,
            *astreams,
            pl.BlockSpec((_BM, H), mrow),
            pl.BlockSpec((_BM, H), mrow),
            pl.BlockSpec((3, H, C), lambda i, k: (0, 0, 0)),
            pl.BlockSpec((1, C), const2),
        ],
        out_specs=pl.BlockSpec((_BM, C), mrow),
        out_shape=jax.ShapeDtypeStruct((N, C), jnp.float32),
        scratch_shapes=[
            pltpu.VMEM((NP, H), jnp.bfloat16),
            pltpu.VMEM((_BM, H), jnp.float32),
            pltpu.SemaphoreType.DMA,
        ],
        compiler_params=_params(2),
    )(th1, *([abf] * _S), hf, hb, W2b, b2r)

    return out


# speedup vs baseline: 1.0062x; 1.0062x over previous
import jax
import jax.numpy as jnp
from jax.experimental import pallas as pl
from jax.experimental.pallas import tpu as pltpu

_BM1 = 200
_BM = 2000
_NK = 2
_BKP = 5120
_S = 4
_BW = _BKP // _S


def _p1_kernel(adj_ref, xb_ref, abf_ref, t1_ref):
    n = adj_ref.shape[1]
    ab = adj_ref[...].astype(jnp.bfloat16)
    abf_ref[:, :n] = ab
    abf_ref[:, n:] = jnp.zeros((abf_ref.shape[0], abf_ref.shape[1] - n),
                               jnp.bfloat16)
    t1_ref[...] = jnp.dot(
        ab, xb_ref[...], preferred_element_type=jnp.float32
    ).astype(jnp.bfloat16)


def _fetch_once(hbm_ref, vmem_ref, sem):
    @pl.when((pl.program_id(0) == 0) & (pl.program_id(1) == 0))
    def _():
        n = hbm_ref.shape[0]
        np_ = vmem_ref.shape[0]
        vmem_ref[pl.ds(n, np_ - n), :] = jnp.zeros(
            (np_ - n, vmem_ref.shape[1]), vmem_ref.dtype)
        cp = pltpu.make_async_copy(hbm_ref, vmem_ref.at[pl.ds(0, n), :], sem)
        cp.start()
        cp.wait()


def _partial(a_refs, vf, k, acc):
    part = jnp.dot(
        a_refs[0][...], vf[pl.ds(k * _BKP, _BW), :],
        preferred_element_type=jnp.float32
    )
    for j in range(1, _S):
        part = part + jnp.dot(
            a_refs[j][...], vf[pl.ds(k * _BKP + j * _BW, _BW), :],
            preferred_element_type=jnp.float32
        )

    @pl.when(k == 0)
    def _():
        acc[...] = part

    @pl.when(k > 0)
    def _():
        acc[...] = acc[...] + part


def _ax_kernel(vf_hbm, *rest):
    a_refs, (o_ref, vf, acc, sem) = rest[:_S], rest[_S:]
    _fetch_once(vf_hbm, vf, sem)
    k = pl.program_id(1)
    _partial(a_refs, vf, k, acc)

    @pl.when(k == _NK - 1)
    def _():
        o_ref[...] = acc[...].astype(jnp.bfloat16)


def _p2_kernel(t1_hbm, *rest):
    a_refs = rest[:_S]
    x_ref, xb_ref, w_ref, b_ref, hf_ref, hb_ref, t1f, acc, sem = rest[_S:]
    _fetch_once(t1_hbm, t1f, sem)
    i, k = pl.program_id(0), pl.program_id(1)
    _partial(a_refs, t1f, k, acc)

    @pl.when(k == _NK - 1)
    def _():
        t1_blk = t1f[pl.ds(i * _BM, _BM), :]
        tx2 = 2.0 * acc[...] - x_ref[...]
        h = (
            jnp.dot(xb_ref[...], w_ref[0, :, :], preferred_element_type=jnp.float32)
            + jnp.dot(t1_blk, w_ref[1, :, :], preferred_element_type=jnp.float32)
            + jnp.dot(tx2.astype(jnp.bfloat16), w_ref[2, :, :],
                      preferred_element_type=jnp.float32)
            + b_ref[...]
        )
        h = jnp.maximum(h, 0.0)
        hf_ref[...] = h
        hb_ref[...] = h.astype(jnp.bfloat16)


def _p4_kernel(t1_hbm, *rest):
    a_refs = rest[:_S]
    hf_ref, hb_ref, w_ref, b_ref, o_ref, t1f, acc, sem = rest[_S:]
    _fetch_once(t1_hbm, t1f, sem)
    i, k = pl.program_id(0), pl.program_id(1)
    _partial(a_refs, t1f, k, acc)

    @pl.when(k == _NK - 1)
    def _():
        t1_blk = t1f[pl.ds(i * _BM, _BM), :]
        th2 = 2.0 * acc[...] - hf_ref[...]
        logits = (
            jnp.dot(hb_ref[...], w_ref[0, :, :], preferred_element_type=jnp.float32)
            + jnp.dot(t1_blk, w_ref[1, :, :], preferred_element_type=jnp.float32)
            + jnp.dot(th2.astype(jnp.bfloat16), w_ref[2, :, :],
                      preferred_element_type=jnp.float32)
            + b_ref[...]
        )
        m = jnp.max(logits, axis=1, keepdims=True)
        e = logits - m
        o_ref[...] = e - jnp.log(jnp.sum(jnp.exp(e), axis=1, keepdims=True))


def _params(n_dims):
    return pltpu.CompilerParams(dimension_semantics=("arbitrary",) * n_dims)


def kernel(x, adj, W1, b1, W2, b2):
    N, F = x.shape
    H = W1.shape[2]
    C = W2.shape[2]
    NP = _NK * _BKP
    xb = x.astype(jnp.bfloat16)
    W1b = W1.astype(jnp.bfloat16)
    W2b = W2.astype(jnp.bfloat16)
    b1r = b1.reshape(1, H)
    b2r = b2.reshape(1, C)
    grid2 = (N // _BM, _NK)

    astreams = [
        pl.BlockSpec((_BM, _BW), (lambda j: (lambda i, k: (i, _S * k + j)))(j))
        for j in range(_S)
    ]
    mrow = lambda i, k: (i, 0)
    const2 = lambda i, k: (0, 0)
    hbm = pl.BlockSpec(memory_space=pl.ANY)

    abf, t1 = pl.pallas_call(
        _p1_kernel,
        grid=(N // _BM1,),
        in_specs=[
            pl.BlockSpec((_BM1, N), lambda i: (i, 0)),
            pl.BlockSpec((N, F), lambda i: (0, 0)),
        ],
        out_specs=[
            pl.BlockSpec((_BM1, NP), lambda i: (i, 0)),
            pl.BlockSpec((_BM1, F), lambda i: (i, 0)),
        ],
        out_shape=[
            jax.ShapeDtypeStruct((N, NP), jnp.bfloat16),
            jax.ShapeDtypeStruct((N, F), jnp.bfloat16),
        ],
        compiler_params=_params(1),
    )(adj, xb)

    hf, hb = pl.pallas_call(
        _p2_kernel,
        grid=grid2,
        in_specs=[
            hbm,
            *astreams,
            pl.BlockSpec((_BM, F), mrow),
            pl.BlockSpec((_BM, F), mrow),
            pl.BlockSpec((3, F, H), lambda i, k: (0, 0, 0)),
            pl.BlockSpec((1, H), const2),
        ],
        out_specs=[
            pl.BlockSpec((_BM, H), mrow),
            pl.BlockSpec((_BM, H), mrow),
        ],
        out_shape=[
            jax.ShapeDtypeStruct((N, H), jnp.float32),
            jax.ShapeDtypeStruct((N, H), jnp.bfloat16),
        ],
        scratch_shapes=[
            pltpu.VMEM((NP, F), jnp.bfloat16),
            pltpu.VMEM((_BM, H), jnp.float32),
            pltpu.SemaphoreType.DMA,
        ],
        compiler_params=_params(2),
    )(t1, *([abf] * _S), x, xb, W1b, b1r)

    th1 = pl.pallas_call(
        _ax_kernel,
        grid=grid2,
        in_specs=[
            hbm,
            *astreams,
        ],
        out_specs=pl.BlockSpec((_BM, H), mrow),
        out_shape=jax.ShapeDtypeStruct((N, H), jnp.bfloat16),
        scratch_shapes=[
            pltpu.VMEM((NP, H), jnp.bfloat16),
            pltpu.VMEM((_BM, H), jnp.float32),
            pltpu.SemaphoreType.DMA,
        ],
        compiler_params=_params(2),
    )(hb, *([abf] * _S))

    out = pl.pallas_call(
        _p4_kernel,
        grid=grid2,
        in_specs=[
            hbm,
            *astreams,
            pl.BlockSpec((_BM, H), mrow),
            pl.BlockSpec((_BM, H), mrow),
            pl.BlockSpec((3, H, C), lambda i, k: (0, 0, 0)),
            pl.BlockSpec((1, C), const2),
        ],
        out_specs=pl.BlockSpec((_BM, C), mrow),
        out_shape=jax.ShapeDtypeStruct((N, C), jnp.float32),
        scratch_shapes=[
            pltpu.VMEM((NP, H), jnp.bfloat16),
            pltpu.VMEM((_BM, H), jnp.float32),
            pltpu.SemaphoreType.DMA,
        ],
        compiler_params=_params(2),
    )(th1, *([abf] * _S), hf, hb, W2b, b2r)

    return out


# speedup vs baseline: 1.0629x; 1.0563x over previous
import jax
import jax.numpy as jnp
from jax.experimental import pallas as pl
from jax.experimental.pallas import tpu as pltpu

_BM1 = 400
_BM = 2000
_NK = 2
_BKP = 5120
_S = 4
_BW = _BKP // _S


def _p1_kernel(adj_ref, xb_ref, abf_ref, t1_ref):
    n = adj_ref.shape[1]
    ab = adj_ref[...].astype(jnp.bfloat16)
    abf_ref[:, :n] = ab
    abf_ref[:, n:] = jnp.zeros((abf_ref.shape[0], abf_ref.shape[1] - n),
                               jnp.bfloat16)
    t1_ref[...] = jnp.dot(
        ab, xb_ref[...], preferred_element_type=jnp.float32
    ).astype(jnp.bfloat16)


def _mega_kernel(t1_hbm, abf_hbm, xb_ref, w1_ref, b1_ref, w2_ref, b2_ref,
                 out_hbm, vt1, vh, acc, sem):
    vu = vt1
    n = t1_hbm.shape[0]
    npad = vt1.shape[0]
    c = out_hbm.shape[1]

    zf = jnp.zeros((npad - n, vt1.shape[1]), jnp.bfloat16)
    vt1[pl.ds(n, npad - n), :] = zf
    vh[pl.ds(n, npad - n), :] = zf
    cp = pltpu.make_async_copy(t1_hbm, vt1.at[pl.ds(0, n), :], sem)
    cp.start()
    cp.wait()

    grid = (n // _BM, _NK)
    a_specs = [
        pl.BlockSpec((_BM, _BW), (lambda j: (lambda i, k: (i, _S * k + j)))(j))
        for j in range(_S)
    ]

    def partial(a_blks, vf, k):
        part = jnp.dot(a_blks[0][...], vf[pl.ds(k * _BKP, _BW), :],
                       preferred_element_type=jnp.float32)
        for j in range(1, _S):
            part = part + jnp.dot(
                a_blks[j][...], vf[pl.ds(k * _BKP + j * _BW, _BW), :],
                preferred_element_type=jnp.float32)

        @pl.when(k == 0)
        def _():
            acc[...] = part

        @pl.when(k > 0)
        def _():
            acc[...] = acc[...] + part

    def body2(idx, *a_blks):
        i, k = idx
        partial(a_blks, vt1, k)

        @pl.when(k == _NK - 1)
        def _():
            rows = pl.ds(i * _BM, _BM)
            xb_blk = xb_ref[rows, :]
            tx2 = 2.0 * acc[...] - xb_blk.astype(jnp.float32)
            h = (
                jnp.dot(xb_blk, w1_ref[0, :, :],
                        preferred_element_type=jnp.float32)
                + jnp.dot(vt1[rows, :], w1_ref[1, :, :],
                          preferred_element_type=jnp.float32)
                + jnp.dot(tx2.astype(jnp.bfloat16), w1_ref[2, :, :],
                          preferred_element_type=jnp.float32)
                + b1_ref[...]
            )
            vh[rows, :] = jnp.maximum(h, 0.0).astype(jnp.bfloat16)

    def body3(idx, *a_blks):
        i, k = idx
        partial(a_blks, vh, k)

        @pl.when(k == _NK - 1)
        def _():
            vu[pl.ds(i * _BM, _BM), :] = acc[...].astype(jnp.bfloat16)

    def body4(idx, *args):
        a_blks, o_blk = args[:_S], args[_S]
        i, k = idx
        partial(a_blks, vu, k)

        @pl.when(k == _NK - 1)
        def _():
            rows = pl.ds(i * _BM, _BM)
            hb_blk = vh[rows, :]
            th2 = 2.0 * acc[...] - hb_blk.astype(jnp.float32)
            logits = (
                jnp.dot(hb_blk, w2_ref[0, :, :],
                        preferred_element_type=jnp.float32)
                + jnp.dot(vu[rows, :], w2_ref[1, :, :],
                          preferred_element_type=jnp.float32)
                + jnp.dot(th2.astype(jnp.bfloat16), w2_ref[2, :, :],
                          preferred_element_type=jnp.float32)
                + b2_ref[...]
            )
            m = jnp.max(logits, axis=1, keepdims=True)
            e = logits - m
            o_blk[...] = e - jnp.log(jnp.sum(jnp.exp(e), axis=1, keepdims=True))

    abf4 = [abf_hbm] * _S
    ospec = [pl.BlockSpec((_BM, c), lambda i, k: (i, 0))]
    pipe2 = pltpu.emit_pipeline(
        body2, grid=grid, in_specs=a_specs, _explicit_indices=True)
    pipe3 = pltpu.emit_pipeline(
        body3, grid=grid, in_specs=a_specs, _explicit_indices=True)
    pipe4 = pltpu.emit_pipeline(
        body4, grid=grid, in_specs=a_specs, out_specs=ospec,
        _explicit_indices=True)
    _, mk_allocs = pltpu.emit_pipeline_with_allocations(
        body4, grid=grid, in_specs=a_specs, out_specs=ospec)

    def run(allocs):
        pipe2(*abf4, allocations=tuple(allocs)[:_S])
        pipe3(*abf4, allocations=tuple(allocs)[:_S])
        pipe4(*abf4, out_hbm, allocations=tuple(allocs))

    pl.run_scoped(run, mk_allocs(*abf4, out_hbm))


def kernel(x, adj, W1, b1, W2, b2):
    N, F = x.shape
    H = W1.shape[2]
    C = W2.shape[2]
    NP = _NK * _BKP
    xb = x.astype(jnp.bfloat16)
    W1b = W1.astype(jnp.bfloat16)
    W2b = W2.astype(jnp.bfloat16)
    b1r = b1.reshape(1, H)
    b2r = b2.reshape(1, C)
    hbm = pl.BlockSpec(memory_space=pl.ANY)

    abf, t1 = pl.pallas_call(
        _p1_kernel,
        grid=(N // _BM1,),
        in_specs=[
            pl.BlockSpec((_BM1, N), lambda i: (i, 0)),
            pl.BlockSpec((N, F), lambda i: (0, 0)),
        ],
        out_specs=[
            pl.BlockSpec((_BM1, NP), lambda i: (i, 0)),
            pl.BlockSpec((_BM1, F), lambda i: (i, 0)),
        ],
        out_shape=[
            jax.ShapeDtypeStruct((N, NP), jnp.bfloat16),
            jax.ShapeDtypeStruct((N, F), jnp.bfloat16),
        ],
        compiler_params=pltpu.CompilerParams(
            dimension_semantics=("arbitrary",)),
    )(adj, xb)

    out = pl.pallas_call(
        _mega_kernel,
        in_specs=[
            hbm,
            hbm,
            pl.BlockSpec((N, F), lambda: (0, 0)),
            pl.BlockSpec((3, F, H), lambda: (0, 0, 0)),
            pl.BlockSpec((1, H), lambda: (0, 0)),
            pl.BlockSpec((3, H, C), lambda: (0, 0, 0)),
            pl.BlockSpec((1, C), lambda: (0, 0)),
        ],
        out_specs=pl.BlockSpec(memory_space=pl.ANY),
        out_shape=jax.ShapeDtypeStruct((N, C), jnp.float32),
        scratch_shapes=[
            pltpu.VMEM((NP, F), jnp.bfloat16),
            pltpu.VMEM((NP, H), jnp.bfloat16),
            pltpu.VMEM((_BM, H), jnp.float32),
            pltpu.SemaphoreType.DMA,
        ],
    )(t1, abf, xb, W1b, b1r, W2b, b2r)

    return out
